# baseline (speedup 1.0000x reference)
import functools

import jax
import jax.numpy as jnp
from jax import lax
from jax.experimental import pallas as pl
from jax.experimental.pallas import tpu as pltpu
from jax.experimental.pallas import tpu_sc as plsc

PAD = 1
L = 16


def _build_sc_call(bsz, seq_len, vocab, dim):
    info = plsc.get_sparse_core_info()
    nw = info.num_cores * info.num_subcores
    chunks_per_row = nw // bsz
    chunk = seq_len // chunks_per_row
    vecs_per_chunk = chunk // L
    guard = chunk // 2
    g_rows = 32
    n_g = chunk // g_rows

    mesh = plsc.VectorSubcoreMesh(core_axis_name="c", subcore_axis_name="s")

    @functools.partial(
        pl.kernel,
        mesh=mesh,
        out_type=jax.ShapeDtypeStruct((bsz * seq_len, dim), jnp.float32),
        scratch_types=[
            pltpu.VMEM((seq_len,), jnp.int32),
            pltpu.VMEM((chunk,), jnp.int32),
            pltpu.VMEM((L,), jnp.int32),
            pltpu.VMEM((3 * L,), jnp.int32),
            pltpu.VMEM((guard + chunk,), jnp.int32),
            pltpu.VMEM((3, g_rows, dim), jnp.float32),
            pltpu.SemaphoreType.DMA,
            pltpu.SemaphoreType.DMA,
        ],
    )
    def sc_kernel(tok_hbm, table_hbm, out_hbm, tok_v, idx_v, acc_v, red_v,
                  cum_v, rows_v, sem_g, sem_w):
        cid = lax.axis_index("c")
        sid = lax.axis_index("s")
        wid = sid * info.num_cores + cid
        row = wid // chunks_per_row
        c = wid % chunks_per_row
        zeros = jnp.zeros((L,), jnp.int32)

        pltpu.sync_copy(tok_hbm.at[pl.ds(row * seq_len, seq_len)], tok_v)

        acc_v[...] = zeros

        def pre_body(i, dummy):
            t = tok_v[pl.ds(i * L, L)]
            acc_v[...] = acc_v[...] + jnp.minimum(jnp.abs(t - PAD), 1)
            return dummy

        lax.fori_loop(0, c * vecs_per_chunk, pre_body, jnp.int32(0))

        red_v[pl.ds(0, L)] = acc_v[...]
        red_v[pl.ds(L, L)] = zeros
        red_v[pl.ds(2 * L, L)] = zeros
        for k in (8, 4, 2, 1):
            red_v[pl.ds(0, L)] = red_v[pl.ds(0, L)] + red_v[pl.ds(k, L)]
        total_vec = red_v[pl.ds(0, L)]
        for j in range(L):
            red_v[pl.ds(L + j, L)] = total_vec
        prefix = red_v[pl.ds(L, L)]

        for j in range(guard // L):
            cum_v[pl.ds(j * L, L)] = zeros
        chunk_off = c * chunk

        def mask_body(j, dummy):
            t = tok_v[pl.ds(chunk_off + j * L, L)]
            cum_v[pl.ds(guard + j * L, L)] = jnp.minimum(jnp.abs(t - PAD), 1)
            return dummy

        lax.fori_loop(0, vecs_per_chunk, mask_body, jnp.int32(0))

        shift = 1
        while shift < chunk:
            def step_body(jr, dummy, k=shift):
                j = vecs_per_chunk - 1 - jr
                base = guard + j * L
                cum_v[pl.ds(base, L)] = (cum_v[pl.ds(base, L)]
                                         + cum_v[pl.ds(base - k, L)])
                return dummy

            lax.fori_loop(0, vecs_per_chunk, step_body, jnp.int32(0))
            shift *= 2

        def pos_body(j, dummy):
            t = tok_v[pl.ds(chunk_off + j * L, L)]
            m = jnp.minimum(jnp.abs(t - PAD), 1)
            cs = cum_v[pl.ds(guard + j * L, L)]
            idx_v[pl.ds(j * L, L)] = (prefix + cs) * m + PAD
            return dummy

        lax.fori_loop(0, vecs_per_chunk, pos_body, jnp.int32(0))

        base = row * seq_len + chunk_off
        nbuf = 3

        def start_gather(g, buf):
            return pltpu.async_copy(
                table_hbm.at[idx_v.at[pl.ds(g * g_rows, g_rows)]],
                rows_v.at[buf], sem_g)

        def start_write(g, buf):
            return pltpu.async_copy(
                rows_v.at[buf], out_hbm.at[pl.ds(base + g * g_rows, g_rows)],
                sem_w)

        pend_g = [None] * nbuf
        pend_w = [None] * nbuf
        for g in range(min(nbuf, n_g)):
            pend_g[g % nbuf] = start_gather(g, g % nbuf)
        for g in range(n_g):
            b = g % nbuf
            pend_g[b].wait()
            pend_w[b] = start_write(g, b)
            nxt = g + nbuf
            if nxt < n_g:
                pend_w[b].wait()
                pend_g[b] = start_gather(nxt, b)
                pend_w[b] = None
        for b in range(nbuf):
            if pend_w[b] is not None:
                pend_w[b].wait()

    return sc_kernel


def kernel(input, weights):
    bsz, seq_len = input.shape
    vocab, dim = weights.shape
    tok = input.astype(jnp.int32).reshape(-1)
    call = _build_sc_call(bsz, seq_len, vocab, dim)
    out = call(tok, weights)
    return out.reshape(bsz, seq_len, dim)
